# baseline (speedup 1.0000x reference)
import functools

import jax
import jax.numpy as jnp
from jax import lax
from jax.experimental import pallas as pl
from jax.experimental.pallas import tpu as pltpu

N = 1024
PACK = 16
NJ = N // PACK
IBLK = 64
GRID = N // IBLK
RB = IBLK * NJ

def _ind_const():
    return jnp.concatenate([
        jnp.kron(jnp.eye(IBLK, dtype=jnp.bfloat16),
                 jnp.ones((NJ, 1), jnp.bfloat16)),
        jnp.tile(jnp.eye(NJ, dtype=jnp.bfloat16), (IBLK, 1)),
    ], axis=1)


def _fold_const(e_out):
    return jnp.tile(jnp.eye(e_out, dtype=jnp.float32), (PACK, 1))


def _sweep_kernel(e_ref, v_ref, vp_ref, u_ref, ind_ref, fold_ref,
                  wee_ref, wer_ref, wes_ref, weu_ref, bet_ref,
                  wnv_ref, wne_ref, wnu_ref, bn_ref,
                  wgu_ref, wgv_ref, wge_ref, bg_ref,
                  eo_ref, vo_ref, uo_ref,
                  rrep_scr, bias_scr, agg_scr,
                  *, act_relu, residual, kin, kout):
    b = pl.program_id(0)
    hi = lax.Precision.HIGHEST

    @pl.when(b == 0)
    def _prologue():
        rrep_scr[...] = jnp.dot(v_ref[...], wer_ref[...],
                                precision=hi).astype(jnp.bfloat16)
        spc = (jnp.dot(vp_ref[...], wes_ref[...], precision=hi)
               + jnp.dot(u_ref[...], weu_ref[...], precision=hi)
               + bet_ref[...])
        bias_scr[pl.ds(IBLK, NJ), :] = spc.astype(jnp.bfloat16)

    bias_scr[pl.ds(0, IBLK), :] = rrep_scr[pl.ds(b * IBLK, IBLK), :]
    x = e_ref[...]
    y2 = (jnp.dot(x, wee_ref[...], preferred_element_type=jnp.float32)
          + jnp.dot(ind_ref[...], bias_scr[...],
                    preferred_element_type=jnp.float32))
    z = y2.reshape(IBLK, NJ, kout)
    if act_relu:
        z = jnp.maximum(z, 0.0)
    agg_scr[pl.ds(b * IBLK, IBLK), :] = z.sum(axis=1)
    z2 = z.reshape(RB, kout)
    if residual:
        eo_ref[...] = x + z2.astype(eo_ref.dtype)
    else:
        eo_ref[...] = z2.astype(eo_ref.dtype)

    @pl.when(b == GRID - 1)
    def _epilogue():
        aggp = agg_scr[...]
        agg = jnp.dot(aggp, fold_ref[...], precision=hi) / float(N)
        esum = jnp.sum(agg, axis=0, keepdims=True) / float(N)
        v = v_ref[...]
        u = u_ref[...]
        dv = (jnp.dot(v, wnv_ref[...], precision=hi)
              + jnp.dot(agg, wne_ref[...], precision=hi)
              + jnp.dot(u, wnu_ref[...], precision=hi)
              + bn_ref[...])
        if act_relu:
            dv = jnp.maximum(dv, 0.0)
        vmean = jnp.mean(dv, axis=0, keepdims=True)
        du = (jnp.dot(u, wgu_ref[...], precision=hi)
              + jnp.dot(vmean, wgv_ref[...], precision=hi)
              + jnp.dot(esum, wge_ref[...], precision=hi)
              + bg_ref[...])
        if act_relu:
            du = jnp.maximum(du, 0.0)
        if residual:
            vo_ref[...] = v + dv
            uo_ref[...] = u + du
        else:
            vo_ref[...] = dv
            uo_ref[...] = du


def _gn_sweep(E, V, u, wp, *, act_relu, residual, e_dtype=jnp.float32):
    kin = E.shape[-1]
    kout = wp['Wee'].shape[-1]
    wee = wp['Wee'].astype(E.dtype)
    n_in = V.shape[-1]
    n_out = wp['Wnv'].shape[-1]
    g_out = wp['Wgu'].shape[-1]
    Vp = V.reshape(NJ, PACK * n_in)
    ind = _ind_const()
    e_out = kout // PACK
    fold = _fold_const(e_out)

    kfn = functools.partial(_sweep_kernel, act_relu=act_relu,
                            residual=residual, kin=kin, kout=kout)
    full = lambda shp: pl.BlockSpec(shp, lambda b: (0,) * len(shp))
    eo, vo, uo = pl.pallas_call(
        kfn,
        grid=(GRID,),
        in_specs=[
            pl.BlockSpec((RB, kin), lambda b: (b, 0)),
            full((N, n_in)),
            full((NJ, PACK * n_in)),
            full((1, u.shape[-1])),
            full((RB, IBLK + NJ)),
            full((kout, e_out)),
            full(wee.shape),
            full(wp['Wer'].shape),
            full(wp['Wes'].shape),
            full(wp['Weu'].shape),
            full(wp['bet'].shape),
            full(wp['Wnv'].shape),
            full(wp['Wne'].shape),
            full(wp['Wnu'].shape),
            full(wp['bn'].shape),
            full(wp['Wgu'].shape),
            full(wp['Wgv'].shape),
            full(wp['Wge'].shape),
            full(wp['bg'].shape),
        ],
        out_specs=[
            pl.BlockSpec((RB, kout), lambda b: (b, 0)),
            full((N, n_out)),
            full((1, g_out)),
        ],
        out_shape=[
            jax.ShapeDtypeStruct((N * NJ, kout), e_dtype),
            jax.ShapeDtypeStruct((N, n_out), jnp.float32),
            jax.ShapeDtypeStruct((1, g_out), jnp.float32),
        ],
        scratch_shapes=[
            pltpu.VMEM((N, kout), jnp.bfloat16),
            pltpu.VMEM((IBLK + NJ, kout), jnp.bfloat16),
            pltpu.VMEM((N, kout), jnp.float32),
        ],
        compiler_params=pltpu.CompilerParams(
            dimension_semantics=("arbitrary",)),
    )(E, V, Vp, u, ind, fold,
      wee, wp['Wer'], wp['Wes'], wp['Weu'], wp['bet'],
      wp['Wnv'], wp['Wne'], wp['Wnu'], wp['bn'],
      wp['Wgu'], wp['Wgv'], wp['Wge'], wp['bg'])
    return eo, vo, uo


def _prep_block(p):
    e_in, e_out = p['We_e'].shape
    eye = jnp.eye(PACK, dtype=jnp.float32)
    return {
        'Wee': jnp.kron(eye, p['We_e']),
        'Wer': jnp.tile(p['We_r'], (1, PACK)),
        'Wes': jnp.kron(eye, p['We_s']),
        'Weu': jnp.tile(p['We_u'], (1, PACK)),
        'bet': jnp.tile(p['be'], PACK)[None, :],
        'Wnv': p['Wn_v'], 'Wne': p['Wn_e'], 'Wnu': p['Wn_u'],
        'bn': p['bn'][None, :],
        'Wgu': p['Wg_u'], 'Wgv': p['Wg_v'], 'Wge': p['Wg_e'],
        'bg': p['bg'][None, :],
    }


def kernel(u, V, A, params):
    e_in = A.shape[-1]
    E = A.reshape(N * NJ, PACK * e_in)
    uc = u[None, :]
    E, V, uc = _gn_sweep(E, V, uc, _prep_block(params['enc']),
                         act_relu=True, residual=False, e_dtype=jnp.bfloat16)
    for p in params['proc']:
        E, V, uc = _gn_sweep(E, V, uc, _prep_block(p),
                             act_relu=True, residual=True, e_dtype=jnp.bfloat16)
    E, V, uc = _gn_sweep(E, V, uc, _prep_block(params['dec']),
                         act_relu=False, residual=False)
    e_out = params['dec']['We_e'].shape[-1]
    return uc[0], V, E.reshape(N, N, e_out)


# speedup vs baseline: 5.5818x; 5.5818x over previous
import functools

import jax
import jax.numpy as jnp
from jax import lax
from jax.experimental import pallas as pl
from jax.experimental.pallas import tpu as pltpu

N = 1024
PACK = 16
NJ = N // PACK
IBLK = 64
GRID = N // IBLK
RB = IBLK * NJ

def _ind_const():
    return jnp.concatenate([
        jnp.kron(jnp.eye(IBLK, dtype=jnp.bfloat16),
                 jnp.ones((NJ, 1), jnp.bfloat16)),
        jnp.tile(jnp.eye(NJ, dtype=jnp.bfloat16), (IBLK, 1)),
    ], axis=1)


def _fold_const(e_out):
    return jnp.tile(jnp.eye(e_out, dtype=jnp.float32), (PACK, 1))


def _sweep_kernel(e_ref, v_ref, vp_ref, u_ref, ind_ref, fold_ref,
                  wee_ref, wer_ref, wes_ref, weu_ref, bet_ref,
                  wnv_ref, wne_ref, wnu_ref, bn_ref,
                  wgu_ref, wgv_ref, wge_ref, bg_ref,
                  eo_ref, vo_ref, uo_ref,
                  rrep_scr, bias_scr, agg_scr,
                  *, act_relu, residual, kin, kout):
    b = pl.program_id(0)
    hi = lax.Precision.HIGHEST

    @pl.when(b == 0)
    def _prologue():
        rrep_scr[...] = jnp.dot(v_ref[...], wer_ref[...],
                                precision=hi).astype(jnp.bfloat16)
        spc = (jnp.dot(vp_ref[...], wes_ref[...], precision=hi)
               + jnp.dot(u_ref[...], weu_ref[...], precision=hi)
               + bet_ref[...])
        bias_scr[pl.ds(IBLK, NJ), :] = spc.astype(jnp.bfloat16)

    bias_scr[pl.ds(0, IBLK), :] = rrep_scr[pl.ds(b * IBLK, IBLK), :]
    x = e_ref[...]
    x2 = x.reshape(RB, kin)
    y2 = (jnp.dot(x2, wee_ref[...], preferred_element_type=jnp.float32)
          + jnp.dot(ind_ref[...], bias_scr[...],
                    preferred_element_type=jnp.float32))
    z = y2.reshape(IBLK, NJ, kout)
    if act_relu:
        z = jnp.maximum(z, 0.0)
    agg_scr[pl.ds(b * IBLK, IBLK), :] = z.sum(axis=1)
    if residual:
        eo_ref[...] = (x.astype(jnp.float32) + z).astype(eo_ref.dtype)
    else:
        eo_ref[...] = z.astype(eo_ref.dtype)

    @pl.when(b == GRID - 1)
    def _epilogue():
        aggp = agg_scr[...]
        agg = jnp.dot(aggp, fold_ref[...], precision=hi) / float(N)
        esum = jnp.sum(agg, axis=0, keepdims=True) / float(N)
        v = v_ref[...]
        u = u_ref[...]
        dv = (jnp.dot(v, wnv_ref[...], precision=hi)
              + jnp.dot(agg, wne_ref[...], precision=hi)
              + jnp.dot(u, wnu_ref[...], precision=hi)
              + bn_ref[...])
        if act_relu:
            dv = jnp.maximum(dv, 0.0)
        vmean = jnp.mean(dv, axis=0, keepdims=True)
        du = (jnp.dot(u, wgu_ref[...], precision=hi)
              + jnp.dot(vmean, wgv_ref[...], precision=hi)
              + jnp.dot(esum, wge_ref[...], precision=hi)
              + bg_ref[...])
        if act_relu:
            du = jnp.maximum(du, 0.0)
        if residual:
            vo_ref[...] = v + dv
            uo_ref[...] = u + du
        else:
            vo_ref[...] = dv
            uo_ref[...] = du


def _gn_sweep(E, V, u, wp, *, act_relu, residual, e_dtype=jnp.float32):
    kin = E.shape[-1]
    kout = wp['Wee'].shape[-1]
    wee = wp['Wee'].astype(E.dtype)
    n_in = V.shape[-1]
    n_out = wp['Wnv'].shape[-1]
    g_out = wp['Wgu'].shape[-1]
    Vp = V.reshape(NJ, PACK * n_in)
    ind = _ind_const()
    e_out = kout // PACK
    fold = _fold_const(e_out)

    kfn = functools.partial(_sweep_kernel, act_relu=act_relu,
                            residual=residual, kin=kin, kout=kout)
    full = lambda shp: pl.BlockSpec(shp, lambda b: (0,) * len(shp))
    eo, vo, uo = pl.pallas_call(
        kfn,
        grid=(GRID,),
        in_specs=[
            pl.BlockSpec((IBLK, NJ, kin), lambda b: (b, 0, 0)),
            full((N, n_in)),
            full((NJ, PACK * n_in)),
            full((1, u.shape[-1])),
            full((RB, IBLK + NJ)),
            full((kout, e_out)),
            full(wee.shape),
            full(wp['Wer'].shape),
            full(wp['Wes'].shape),
            full(wp['Weu'].shape),
            full(wp['bet'].shape),
            full(wp['Wnv'].shape),
            full(wp['Wne'].shape),
            full(wp['Wnu'].shape),
            full(wp['bn'].shape),
            full(wp['Wgu'].shape),
            full(wp['Wgv'].shape),
            full(wp['Wge'].shape),
            full(wp['bg'].shape),
        ],
        out_specs=[
            pl.BlockSpec((IBLK, NJ, kout), lambda b: (b, 0, 0)),
            full((N, n_out)),
            full((1, g_out)),
        ],
        out_shape=[
            jax.ShapeDtypeStruct((N, NJ, kout), e_dtype),
            jax.ShapeDtypeStruct((N, n_out), jnp.float32),
            jax.ShapeDtypeStruct((1, g_out), jnp.float32),
        ],
        scratch_shapes=[
            pltpu.VMEM((N, kout), jnp.bfloat16),
            pltpu.VMEM((IBLK + NJ, kout), jnp.bfloat16),
            pltpu.VMEM((N, kout), jnp.float32),
        ],
        compiler_params=pltpu.CompilerParams(
            dimension_semantics=("arbitrary",)),
    )(E, V, Vp, u, ind, fold,
      wee, wp['Wer'], wp['Wes'], wp['Weu'], wp['bet'],
      wp['Wnv'], wp['Wne'], wp['Wnu'], wp['bn'],
      wp['Wgu'], wp['Wgv'], wp['Wge'], wp['bg'])
    return eo, vo, uo


def _prep_block(p):
    e_in, e_out = p['We_e'].shape
    eye = jnp.eye(PACK, dtype=jnp.float32)
    return {
        'Wee': jnp.kron(eye, p['We_e']),
        'Wer': jnp.tile(p['We_r'], (1, PACK)),
        'Wes': jnp.kron(eye, p['We_s']),
        'Weu': jnp.tile(p['We_u'], (1, PACK)),
        'bet': jnp.tile(p['be'], PACK)[None, :],
        'Wnv': p['Wn_v'], 'Wne': p['Wn_e'], 'Wnu': p['Wn_u'],
        'bn': p['bn'][None, :],
        'Wgu': p['Wg_u'], 'Wgv': p['Wg_v'], 'Wge': p['Wg_e'],
        'bg': p['bg'][None, :],
    }


def kernel(u, V, A, params):
    e_in = A.shape[-1]
    E = A.reshape(N, NJ, PACK * e_in)
    uc = u[None, :]
    E, V, uc = _gn_sweep(E, V, uc, _prep_block(params['enc']),
                         act_relu=True, residual=False, e_dtype=jnp.bfloat16)
    for p in params['proc']:
        E, V, uc = _gn_sweep(E, V, uc, _prep_block(p),
                             act_relu=True, residual=True, e_dtype=jnp.bfloat16)
    E, V, uc = _gn_sweep(E, V, uc, _prep_block(params['dec']),
                         act_relu=False, residual=False)
    e_out = params['dec']['We_e'].shape[-1]
    return uc[0], V, E.reshape(N, N, e_out)


# speedup vs baseline: 6.1336x; 1.0989x over previous
import functools

import jax
import jax.numpy as jnp
from jax import lax
from jax.experimental import pallas as pl
from jax.experimental.pallas import tpu as pltpu

N = 1024
PACK = 16
NJ = N // PACK
IBLK = 64
GRID = N // IBLK
RB = IBLK * NJ

def _ind_const():
    return jnp.concatenate([
        jnp.kron(jnp.eye(IBLK, dtype=jnp.bfloat16),
                 jnp.ones((NJ, 1), jnp.bfloat16)),
        jnp.tile(jnp.eye(NJ, dtype=jnp.bfloat16), (IBLK, 1)),
    ], axis=1)


def _fold_const(e_out):
    return jnp.tile(jnp.eye(e_out, dtype=jnp.float32), (PACK, 1))


def _sweep_kernel(e_ref, v_ref, vp_ref, u_ref, ind_ref, fold_ref,
                  wee_ref, wer_ref, wes_ref, weu_ref, bet_ref,
                  wnv_ref, wne_ref, wnu_ref, bn_ref,
                  wgu_ref, wgv_ref, wge_ref, bg_ref,
                  eo_ref, vo_ref, uo_ref,
                  rrep_scr, bias_scr, agg_scr,
                  *, act_relu, residual, kin, kout):
    b = pl.program_id(0)
    hi = lax.Precision.HIGHEST

    @pl.when(b == 0)
    def _prologue():
        rrep_scr[...] = jnp.dot(v_ref[...], wer_ref[...],
                                precision=hi).astype(jnp.bfloat16)
        spc = (jnp.dot(vp_ref[...], wes_ref[...], precision=hi)
               + jnp.dot(u_ref[...], weu_ref[...], precision=hi)
               + bet_ref[...])
        bias_scr[pl.ds(IBLK, NJ), :] = spc.astype(jnp.bfloat16)

    bias_scr[pl.ds(0, IBLK), :] = rrep_scr[pl.ds(b * IBLK, IBLK), :]
    x = e_ref[...]
    x2 = x.reshape(RB, kin)
    y2 = (jnp.dot(x2, wee_ref[...], preferred_element_type=jnp.float32)
          + jnp.dot(ind_ref[...], bias_scr[...],
                    preferred_element_type=jnp.float32))
    z = y2.reshape(IBLK, NJ, kout)
    if act_relu:
        z = jnp.maximum(z, 0.0)
    agg_scr[pl.ds(b * IBLK, IBLK), :] = z.sum(axis=1)
    if residual:
        eo_ref[...] = (x.astype(jnp.float32) + z).astype(eo_ref.dtype)
    else:
        eo_ref[...] = z.astype(eo_ref.dtype)

    @pl.when(b == GRID - 1)
    def _epilogue():
        aggp = agg_scr[...]
        agg = jnp.dot(aggp, fold_ref[...], precision=hi) / float(N)
        esum = jnp.sum(agg, axis=0, keepdims=True) / float(N)
        v = v_ref[...]
        u = u_ref[...]
        dv = (jnp.dot(v, wnv_ref[...], precision=hi)
              + jnp.dot(agg, wne_ref[...], precision=hi)
              + jnp.dot(u, wnu_ref[...], precision=hi)
              + bn_ref[...])
        if act_relu:
            dv = jnp.maximum(dv, 0.0)
        vmean = jnp.mean(dv, axis=0, keepdims=True)
        du = (jnp.dot(u, wgu_ref[...], precision=hi)
              + jnp.dot(vmean, wgv_ref[...], precision=hi)
              + jnp.dot(esum, wge_ref[...], precision=hi)
              + bg_ref[...])
        if act_relu:
            du = jnp.maximum(du, 0.0)
        if residual:
            vo_ref[...] = v + dv
            uo_ref[...] = u + du
        else:
            vo_ref[...] = dv
            uo_ref[...] = du


def _gn_sweep(E, V, u, wp, *, act_relu, residual, e_dtype=jnp.float32):
    kin = E.shape[-1]
    kout = wp['Wee'].shape[-1]
    wee = wp['Wee'].astype(E.dtype)
    n_in = V.shape[-1]
    n_out = wp['Wnv'].shape[-1]
    g_out = wp['Wgu'].shape[-1]
    Vp = V.reshape(NJ, PACK * n_in)
    ind = _ind_const()
    e_out = kout // PACK
    fold = _fold_const(e_out)

    kfn = functools.partial(_sweep_kernel, act_relu=act_relu,
                            residual=residual, kin=kin, kout=kout)
    full = lambda shp: pl.BlockSpec(shp, lambda b: (0,) * len(shp))
    eo, vo, uo = pl.pallas_call(
        kfn,
        grid=(GRID,),
        in_specs=[
            pl.BlockSpec((IBLK, NJ, kin), lambda b: (b, 0, 0)),
            full((N, n_in)),
            full((NJ, PACK * n_in)),
            full((1, u.shape[-1])),
            full((RB, IBLK + NJ)),
            full((kout, e_out)),
            full(wee.shape),
            full(wp['Wer'].shape),
            full(wp['Wes'].shape),
            full(wp['Weu'].shape),
            full(wp['bet'].shape),
            full(wp['Wnv'].shape),
            full(wp['Wne'].shape),
            full(wp['Wnu'].shape),
            full(wp['bn'].shape),
            full(wp['Wgu'].shape),
            full(wp['Wgv'].shape),
            full(wp['Wge'].shape),
            full(wp['bg'].shape),
        ],
        out_specs=[
            pl.BlockSpec((IBLK, NJ, kout), lambda b: (b, 0, 0)),
            full((N, n_out)),
            full((1, g_out)),
        ],
        out_shape=[
            jax.ShapeDtypeStruct((N, NJ, kout), e_dtype),
            jax.ShapeDtypeStruct((N, n_out), jnp.float32),
            jax.ShapeDtypeStruct((1, g_out), jnp.float32),
        ],
        scratch_shapes=[
            pltpu.VMEM((N, kout), jnp.bfloat16),
            pltpu.VMEM((IBLK + NJ, kout), jnp.bfloat16),
            pltpu.VMEM((N, kout), jnp.float32),
        ],
        compiler_params=pltpu.CompilerParams(
            dimension_semantics=("arbitrary",)),
    )(E, V, Vp, u, ind, fold,
      wee, wp['Wer'], wp['Wes'], wp['Weu'], wp['bet'],
      wp['Wnv'], wp['Wne'], wp['Wnu'], wp['bn'],
      wp['Wgu'], wp['Wgv'], wp['Wge'], wp['bg'])
    return eo, vo, uo


def _prep_block(p):
    e_in, e_out = p['We_e'].shape
    eye = jnp.eye(PACK, dtype=jnp.float32)
    return {
        'Wee': jnp.kron(eye, p['We_e']),
        'Wer': jnp.tile(p['We_r'], (1, PACK)),
        'Wes': jnp.kron(eye, p['We_s']),
        'Weu': jnp.tile(p['We_u'], (1, PACK)),
        'bet': jnp.tile(p['be'], PACK)[None, :],
        'Wnv': p['Wn_v'], 'Wne': p['Wn_e'], 'Wnu': p['Wn_u'],
        'bn': p['bn'][None, :],
        'Wgu': p['Wg_u'], 'Wgv': p['Wg_v'], 'Wge': p['Wg_e'],
        'bg': p['bg'][None, :],
    }


NH = 32
GH = 32
KH = 256


def _mega_kernel(a_ref, v0_ref, u0_ref, ind_ref, fold_ref,
                 t16_ref, m16_ref, sel2_ref,
                 wee0_ref, weeS_ref, werS_ref, wesS_ref, weuS_ref, betS_ref,
                 wnvS_ref, wneS_ref, wnuS_ref, bnS_ref,
                 wguS_ref, wgvS_ref, wgeS_ref, bgS_ref,
                 eo_ref, vo_ref, uo_ref,
                 e_scr, v_scr, u_scr, rrep_scr, bias_scr, agg_scr,
                 *, n_sweeps):
    s = pl.program_id(0)
    b = pl.program_id(1)
    hi = lax.Precision.HIGHEST
    sm1 = jnp.maximum(s - 1, 0)

    @pl.when(jnp.logical_and(s == 0, b == 0))
    def _init():
        v_scr[...] = v0_ref[...]
        u_scr[...] = u0_ref[...]

    @pl.when(b == 0)
    def _sweep_prologue():
        v = v_scr[...]
        u = u_scr[...]
        rrep_scr[...] = jnp.dot(v, werS_ref[s],
                                precision=hi).astype(jnp.bfloat16)
        sv = jnp.dot(v, wesS_ref[s], precision=hi)
        g = jnp.dot(sv, t16_ref[...], precision=hi) * m16_ref[...]
        spc = (jnp.dot(sel2_ref[...], g, precision=hi)
               + jnp.dot(u, weuS_ref[s], precision=hi)
               + betS_ref[s])
        bias_scr[pl.ds(IBLK, NJ), :] = spc.astype(jnp.bfloat16)

    bias_scr[pl.ds(0, IBLK), :] = rrep_scr[pl.ds(b * IBLK, IBLK), :]
    bias = jnp.dot(ind_ref[...], bias_scr[...],
                   preferred_element_type=jnp.float32)

    def _edge_stage(y2, write_escr, write_out, residual, x_res):
        z = y2.reshape(IBLK, NJ, KH)
        z = jnp.maximum(z, 0.0)
        agg_scr[pl.ds(b * IBLK, IBLK), :] = z.sum(axis=1)
        zb = z.astype(jnp.bfloat16)
        if residual:
            zb = x_res + zb
        if write_escr:
            e_scr[pl.ds(b * IBLK, IBLK), :, :] = zb
        if write_out:
            eo_ref[...] = zb

    @pl.when(s == 0)
    def _enc_step():
        x2 = a_ref[...].reshape(RB, a_ref.shape[-1])
        y2 = jnp.dot(x2, wee0_ref[...],
                     preferred_element_type=jnp.float32) + bias
        _edge_stage(y2, True, False, False, None)

    @pl.when(jnp.logical_and(s > 0, s < n_sweeps - 1))
    def _proc_step():
        x = e_scr[pl.ds(b * IBLK, IBLK), :, :]
        y2 = jnp.dot(x.reshape(RB, KH), weeS_ref[sm1],
                     preferred_element_type=jnp.float32) + bias
        _edge_stage(y2, True, False, True, x)

    @pl.when(s == n_sweeps - 1)
    def _last_step():
        x = e_scr[pl.ds(b * IBLK, IBLK), :, :]
        y2 = jnp.dot(x.reshape(RB, KH), weeS_ref[sm1],
                     preferred_element_type=jnp.float32) + bias
        _edge_stage(y2, False, True, True, x)

    @pl.when(b == GRID - 1)
    def _sweep_epilogue():
        aggp = agg_scr[...]
        agg = jnp.dot(aggp, fold_ref[...], precision=hi) / float(N)
        esum = jnp.sum(agg, axis=0, keepdims=True) / float(N)
        v = v_scr[...]
        u = u_scr[...]
        dv = (jnp.dot(v, wnvS_ref[s], precision=hi)
              + jnp.dot(agg, wneS_ref[s], precision=hi)
              + jnp.dot(u, wnuS_ref[s], precision=hi)
              + bnS_ref[s])
        dv = jnp.maximum(dv, 0.0)
        vmean = jnp.mean(dv, axis=0, keepdims=True)
        du = (jnp.dot(u, wguS_ref[s], precision=hi)
              + jnp.dot(vmean, wgvS_ref[s], precision=hi)
              + jnp.dot(esum, wgeS_ref[s], precision=hi)
              + bgS_ref[s])
        du = jnp.maximum(du, 0.0)

        @pl.when(s == 0)
        def _set():
            v_scr[...] = dv
            u_scr[...] = du

        @pl.when(s > 0)
        def _acc():
            v_scr[...] = v + dv
            u_scr[...] = u + du

        @pl.when(s == n_sweeps - 1)
        def _emit():
            vo_ref[...] = v_scr[...]
            uo_ref[...] = u_scr[...]


def _pad_rows(w, rows):
    return jnp.pad(w, ((0, rows - w.shape[0]), (0, 0)))


def _mega(u, V, A, params):
    enc = params['enc']
    procs = params['proc']
    n_sweeps = 1 + len(procs)
    eye = jnp.eye(PACK, dtype=jnp.float32)
    e_in = A.shape[-1]

    wee0 = jnp.kron(eye, enc['We_e'])
    weeS = jnp.stack([jnp.kron(eye, p['We_e'])
                      for p in procs]).astype(jnp.bfloat16)
    werS = jnp.stack([_pad_rows(jnp.tile(p['We_r'], (1, PACK)), NH)
                      for p in [enc] + procs])
    wesS = jnp.stack([_pad_rows(p['We_s'], NH) for p in [enc] + procs])
    weuS = jnp.stack([_pad_rows(jnp.tile(p['We_u'], (1, PACK)), GH)
                      for p in [enc] + procs])
    betS = jnp.stack([jnp.tile(p['be'], PACK)[None, :] for p in [enc] + procs])
    wnvS = jnp.stack([_pad_rows(p['Wn_v'], NH) for p in [enc] + procs])
    wneS = jnp.stack([p['Wn_e'] for p in [enc] + procs])
    wnuS = jnp.stack([_pad_rows(p['Wn_u'], GH) for p in [enc] + procs])
    bnS = jnp.stack([p['bn'][None, :] for p in [enc] + procs])
    wguS = jnp.stack([_pad_rows(p['Wg_u'], GH) for p in [enc] + procs])
    wgvS = jnp.stack([p['Wg_v'] for p in [enc] + procs])
    wgeS = jnp.stack([p['Wg_e'] for p in [enc] + procs])
    bgS = jnp.stack([p['bg'][None, :] for p in [enc] + procs])

    ind = _ind_const()
    fold = _fold_const(PACK)
    lane = jnp.arange(KH)
    t16 = (lane[None, :] % PACK == jnp.arange(PACK)[:, None]
           ).astype(jnp.float32)
    j = jnp.arange(N)
    m16 = (j[:, None] % PACK == lane[None, :] // PACK).astype(jnp.float32)
    sel2 = (j[None, :] // PACK == jnp.arange(NJ)[:, None]
            ).astype(jnp.float32)

    A3 = A.reshape(N, NJ, PACK * e_in)
    V0 = jnp.pad(V, ((0, 0), (0, NH - V.shape[-1])))
    u0 = jnp.pad(u[None, :], ((0, 0), (0, GH - u.shape[-1])))

    kfn = functools.partial(_mega_kernel, n_sweeps=n_sweeps)
    full = lambda shp: pl.BlockSpec(shp, lambda s, b: (0,) * len(shp))
    eo, vo, uo = pl.pallas_call(
        kfn,
        grid=(n_sweeps, GRID),
        in_specs=[
            pl.BlockSpec((IBLK, NJ, PACK * e_in),
                         lambda s, b: (jnp.where(s == 0, b, 0), 0, 0)),
            full((N, NH)),
            full((1, GH)),
            full((RB, IBLK + NJ)),
            full((KH, PACK)),
            full((PACK, KH)),
            full((N, KH)),
            full((NJ, N)),
            full(wee0.shape),
            full(weeS.shape),
            full(werS.shape),
            full(wesS.shape),
            full(weuS.shape),
            full(betS.shape),
            full(wnvS.shape),
            full(wneS.shape),
            full(wnuS.shape),
            full(bnS.shape),
            full(wguS.shape),
            full(wgvS.shape),
            full(wgeS.shape),
            full(bgS.shape),
        ],
        out_specs=[
            pl.BlockSpec((IBLK, NJ, KH),
                         lambda s, b: (jnp.where(s == n_sweeps - 1, b, 0), 0, 0)),
            full((N, NH)),
            full((1, GH)),
        ],
        out_shape=[
            jax.ShapeDtypeStruct((N, NJ, KH), jnp.bfloat16),
            jax.ShapeDtypeStruct((N, NH), jnp.float32),
            jax.ShapeDtypeStruct((1, GH), jnp.float32),
        ],
        scratch_shapes=[
            pltpu.VMEM((N, NJ, KH), jnp.bfloat16),
            pltpu.VMEM((N, NH), jnp.float32),
            pltpu.VMEM((1, GH), jnp.float32),
            pltpu.VMEM((N, KH), jnp.bfloat16),
            pltpu.VMEM((IBLK + NJ, KH), jnp.bfloat16),
            pltpu.VMEM((N, KH), jnp.float32),
        ],
        compiler_params=pltpu.CompilerParams(
            dimension_semantics=("arbitrary", "arbitrary"),
            vmem_limit_bytes=64 * 1024 * 1024),
    )(A3, V0, u0, ind, fold, t16, m16, sel2,
      wee0, weeS, werS, wesS, weuS, betS,
      wnvS, wneS, wnuS, bnS, wguS, wgvS, wgeS, bgS)
    return eo, vo, uo


def kernel(u, V, A, params):
    E, Vh, uh = _mega(u, V, A, params)
    dec = params['dec']
    e_out = dec['We_e'].shape[-1]
    Eo, Vo, uo = _gn_sweep(E, Vh, uh, _prep_block(dec),
                           act_relu=False, residual=False)
    return uo[0], Vo, Eo.reshape(N, N, e_out)
